# Initial kernel scaffold; baseline (speedup 1.0000x reference)
import jax
import jax.numpy as jnp
from jax.experimental import pallas as pl


def kernel(tensor_demo, tensor_med, tensor_vitals, tensor_labs, table, W, b):
    raise NotImplementedError("write your pallas kernel here")


# speedup vs baseline: 1.5573x; 1.5573x over previous
import functools

import jax
import jax.numpy as jnp
from jax import lax
from jax.experimental import pallas as pl
from jax.experimental.pallas import tpu as pltpu
from jax.experimental.pallas import tpu_sc as plsc

D = 128
NC = 2
NS = 16
NW = NC * NS


def _gather_body(idx_hbm, out_hbm, table_hbm, idx_v, rows_v, sem, wid,
                 n_per_tile, chunk, n_chunks):
    base = wid * n_per_tile

    def one_chunk(off):
        pltpu.sync_copy(idx_hbm.at[pl.ds(off, chunk)], idx_v)
        pltpu.async_copy(table_hbm.at[idx_v], rows_v, sem).wait()
        pltpu.sync_copy(rows_v, out_hbm.at[pl.ds(off, chunk)])

    if n_chunks <= 2:
        for k in range(n_chunks):
            one_chunk(base + k * chunk)
    else:
        def body(j, carry):
            one_chunk(base + j * chunk)
            return carry
        lax.fori_loop(0, n_chunks, body, 0)


def _make_sc_gather(n_demo, n_big):
    demo_per = n_demo // NW
    big_per = n_big // NW
    mesh = plsc.VectorSubcoreMesh(core_axis_name="c", subcore_axis_name="s")

    @functools.partial(
        pl.kernel,
        mesh=mesh,
        out_type=[
            jax.ShapeDtypeStruct((n_demo, D), jnp.float32),
            jax.ShapeDtypeStruct((n_big, D), jnp.float32),
            jax.ShapeDtypeStruct((n_big, D), jnp.float32),
            jax.ShapeDtypeStruct((n_big, D), jnp.float32),
        ],
        scratch_types=[
            pltpu.VMEM((448,), jnp.int32),
            pltpu.VMEM((448, D), jnp.float32),
            pltpu.VMEM((400,), jnp.int32),
            pltpu.VMEM((400, D), jnp.float32),
            pltpu.SemaphoreType.DMA,
        ],
    )
    def sc_gather(idx_demo, idx_med, idx_vit, idx_lab, table,
                  out_demo, out_med, out_vit, out_lab,
                  idx_a, rows_a, idx_b, rows_b, sem):
        wid = lax.axis_index("s") * NC + lax.axis_index("c")
        _gather_body(idx_demo, out_demo, table, idx_a, rows_a, sem, wid,
                     demo_per, 448, 2)
        _gather_body(idx_med, out_med, table, idx_b, rows_b, sem, wid,
                     big_per, 400, 16)
        _gather_body(idx_vit, out_vit, table, idx_b, rows_b, sem, wid,
                     big_per, 400, 16)
        _gather_body(idx_lab, out_lab, table, idx_b, rows_b, sem, wid,
                     big_per, 400, 16)

    return sc_gather


def _proj_body(x_ref, wt_ref, b_ref, o_ref):
    o_ref[...] = (
        jnp.dot(jnp.maximum(x_ref[...], 0.0), wt_ref[...],
                preferred_element_type=jnp.float32)
        + b_ref[...]
    )


def _project(x_flat, wt, b2, block_m):
    n = x_flat.shape[0]
    grid = (n // block_m,)
    return pl.pallas_call(
        _proj_body,
        grid=grid,
        in_specs=[
            pl.BlockSpec((block_m, D), lambda i: (i, 0)),
            pl.BlockSpec((D, D), lambda i: (0, 0)),
            pl.BlockSpec((1, D), lambda i: (0, 0)),
        ],
        out_specs=pl.BlockSpec((block_m, D), lambda i: (i, 0)),
        out_shape=jax.ShapeDtypeStruct((n, D), jnp.float32),
    )(x_flat, wt, b2)


def kernel(tensor_demo, tensor_med, tensor_vitals, tensor_labs, table, W, b):
    B, T_demo = tensor_demo.shape
    T_big = tensor_med.shape[1]
    n_demo = B * T_demo
    n_big = B * T_big

    idx_demo = tensor_demo.reshape(-1).astype(jnp.int32)
    idx_med = tensor_med.reshape(-1).astype(jnp.int32)
    idx_vit = tensor_vitals.reshape(-1).astype(jnp.int32)
    idx_lab = tensor_labs.reshape(-1).astype(jnp.int32)

    sc_gather = _make_sc_gather(n_demo, n_big)
    emb_demo_f, emb_med_f, emb_vit_f, emb_lab_f = sc_gather(
        idx_demo, idx_med, idx_vit, idx_lab, table)

    wt = W.T
    b2 = b.reshape(1, D)
    proj_demo_f = _project(emb_demo_f, wt, b2, 2048)
    proj_med_f = _project(emb_med_f, wt, b2, 2048)
    proj_vit_f = _project(emb_vit_f, wt, b2, 2048)
    proj_lab_f = _project(emb_lab_f, wt, b2, 2048)

    emb_demo = emb_demo_f.reshape(B, T_demo, D)
    emb_med = emb_med_f.reshape(B, T_big, D)
    emb_vit = emb_vit_f.reshape(B, T_big, D)
    emb_lab = emb_lab_f.reshape(B, T_big, D)
    proj_demo = proj_demo_f.reshape(B, T_demo, D)
    proj_med = proj_med_f.reshape(B, T_big, D)
    proj_vit = proj_vit_f.reshape(B, T_big, D)
    proj_lab = proj_lab_f.reshape(B, T_big, D)

    embedding = (emb_demo, emb_med, emb_vit, emb_lab)
    projection = (proj_demo, proj_med, proj_vit, proj_lab)
    return (embedding, projection, embedding, projection)
